# baseline (speedup 1.0000x reference)
import functools

import jax
import jax.numpy as jnp
from jax import lax
from jax.experimental import pallas as pl
from jax.experimental.pallas import tpu as pltpu
from jax.experimental.pallas import tpu_sc as plsc

N_NODES = 10000
DIM = 128
N_EDGES = 320000
N_REL = 16

NC = 2
NS = 16
NW = NC * NS

CHUNK = 128
KCH = 80
HKCH = KCH // 2
PW = KCH * CHUNK
E_PAD = PW * NW
NP = 10112
STRIPE = NP // NS

_mesh = plsc.VectorSubcoreMesh(core_axis_name="c", subcore_axis_name="s",
                               num_cores=NC, num_subcores=NS)


def _sc_edge_body(xr_hbm, idx_hbm, dst_hbm, zeros_hbm, agg_out, agg_sh,
                  idx_v, dst_v, buf_a, buf_b, sem_a, sem_b):
    c = lax.axis_index("c")
    s = lax.axis_index("s")
    wid = c * NS + s

    row0 = s * STRIPE
    pltpu.sync_copy(zeros_hbm.at[pl.ds(row0, STRIPE)],
                    agg_sh.at[pl.ds(row0, STRIPE)])
    plsc.subcore_barrier()

    def gat(jv, buf, sem):
        pltpu.async_copy(xr_hbm.at[idx_v.at[jv]], buf, sem)

    def gwait(buf, sem):
        pltpu.make_async_copy(zeros_hbm.at[pl.ds(0, CHUNK)], buf, sem).wait()

    def sca(jv, buf):
        pltpu.sync_copy(buf, agg_sh.at[dst_v.at[jv]], add=True)

    for h in range(2):
        pltpu.sync_copy(idx_hbm.at[wid, h], idx_v)
        pltpu.sync_copy(dst_hbm.at[wid, h], dst_v)
        gat(0, buf_a, sem_a)

        @pl.loop(0, HKCH // 2 - 1)
        def _(p):
            j = 2 * p
            gat(j + 1, buf_b, sem_b)
            gwait(buf_a, sem_a)
            sca(j, buf_a)
            gat(j + 2, buf_a, sem_a)
            gwait(buf_b, sem_b)
            sca(j + 1, buf_b)

        gat(HKCH - 1, buf_b, sem_b)
        gwait(buf_a, sem_a)
        sca(HKCH - 2, buf_a)
        gwait(buf_b, sem_b)
        sca(HKCH - 1, buf_b)

    plsc.subcore_barrier()
    pltpu.sync_copy(agg_sh.at[pl.ds(row0, STRIPE)],
                    agg_out.at[c, pl.ds(row0, STRIPE)])


_sc_edges = pl.kernel(
    _sc_edge_body,
    out_type=jax.ShapeDtypeStruct((NC, NP, DIM), jnp.float32),
    mesh=_mesh,
    scratch_types=(
        pltpu.VMEM_SHARED((NP, DIM), jnp.float32),
        pltpu.VMEM((HKCH, CHUNK), jnp.int32),
        pltpu.VMEM((HKCH, CHUNK), jnp.int32),
        pltpu.VMEM((CHUNK, DIM), jnp.float32),
        pltpu.VMEM((CHUNK, DIM), jnp.float32),
        pltpu.SemaphoreType.DMA,
        pltpu.SemaphoreType.DMA,
    ),
)


def _sc_deg_body(dst_hbm, zdeg_hbm, ones_hbm, deg_out, deg_sh, dst_v,
                 ones_v, sem):
    c = lax.axis_index("c")
    s = lax.axis_index("s")
    wid = c * NS + s

    pltpu.sync_copy(ones_hbm, ones_v)
    row0 = s * STRIPE
    pltpu.sync_copy(zdeg_hbm.at[pl.ds(row0, STRIPE)],
                    deg_sh.at[pl.ds(row0, STRIPE)])
    plsc.subcore_barrier()

    def swait():
        pltpu.make_async_copy(zdeg_hbm.at[pl.ds(0, CHUNK)], ones_v,
                              sem).wait()

    for h in range(2):
        pltpu.sync_copy(dst_hbm.at[wid, h], dst_v)
        pltpu.async_copy(ones_v, deg_sh.at[dst_v.at[0]], sem, add=True)

        @pl.loop(1, HKCH)
        def _(j):
            pltpu.async_copy(ones_v, deg_sh.at[dst_v.at[j]], sem, add=True)
            swait()

        swait()

    plsc.subcore_barrier()
    pltpu.sync_copy(deg_sh.at[pl.ds(row0, STRIPE)],
                    deg_out.at[c, pl.ds(row0, STRIPE)])


_sc_deg = pl.kernel(
    _sc_deg_body,
    out_type=jax.ShapeDtypeStruct((NC, NP, DIM), jnp.float32),
    mesh=_mesh,
    scratch_types=(
        pltpu.VMEM_SHARED((NP, DIM), jnp.float32),
        pltpu.VMEM((HKCH, CHUNK), jnp.int32),
        pltpu.VMEM((CHUNK, DIM), jnp.float32),
        pltpu.SemaphoreType.DMA,
    ),
)


_EROWS = E_PAD // 128


def _eidx_body(src_ref, et_ref, out_ref):
    out_ref[...] = et_ref[...] * N_NODES + src_ref[...]


def _edge_idx(src_p, et_p):
    return pl.pallas_call(
        _eidx_body,
        out_shape=jax.ShapeDtypeStruct((_EROWS, 128), jnp.int32),
    )(src_p.reshape(_EROWS, 128), et_p.reshape(_EROWS, 128))


NB = 400


def _xr_body(x_ref, rel_ref, out_ref):
    out_ref[...] = rel_ref[...][:, None, :] * x_ref[...][None, :, :]


def _build_xr(x, rel):
    grid = (N_NODES // NB,)
    xr = pl.pallas_call(
        _xr_body,
        grid=grid,
        in_specs=[
            pl.BlockSpec((NB, DIM), lambda i: (i, 0)),
            pl.BlockSpec((N_REL, DIM), lambda i: (0, 0)),
        ],
        out_specs=pl.BlockSpec((N_REL, NB, DIM), lambda i: (0, i, 0)),
        out_shape=jax.ShapeDtypeStruct((N_REL, N_NODES, DIM), jnp.float32),
    )(x, rel)
    return xr.reshape(N_REL * N_NODES, DIM)


def _post_body(final, parts_ref, degp_ref, x_ref, W_ref, b_ref, res_ref,
               W2_ref, b2_ref, out_ref):
    deg = degp_ref[0, :, 0:1] + degp_ref[1, :, 0:1]
    deg = jnp.maximum(deg, 1.0)
    agg = (parts_ref[0] + parts_ref[1]) / deg
    h = jax.nn.relu(
        jnp.dot(agg, W_ref[...], preferred_element_type=jnp.float32)
        + b_ref[...])
    emb = x_ref[...] + res_ref[0, 0] * h
    if final:
        out_ref[...] = (
            jnp.dot(emb, W2_ref[...], preferred_element_type=jnp.float32)
            + b2_ref[...])
    else:
        out_ref[...] = emb


def _post(final, parts, degp, x, W, b, res, W2, b2):
    grid = (N_NODES // NB,)
    return pl.pallas_call(
        functools.partial(_post_body, final),
        grid=grid,
        in_specs=[
            pl.BlockSpec((NC, NB, DIM), lambda i: (0, i, 0)),
            pl.BlockSpec((NC, NB, DIM), lambda i: (0, i, 0)),
            pl.BlockSpec((NB, DIM), lambda i: (i, 0)),
            pl.BlockSpec((DIM, DIM), lambda i: (0, 0)),
            pl.BlockSpec((1, DIM), lambda i: (0, 0)),
            pl.BlockSpec((1, 1), lambda i: (0, 0)),
            pl.BlockSpec((DIM, DIM), lambda i: (0, 0)),
            pl.BlockSpec((1, DIM), lambda i: (0, 0)),
        ],
        out_specs=pl.BlockSpec((NB, DIM), lambda i: (i, 0)),
        out_shape=jax.ShapeDtypeStruct((N_NODES, DIM), jnp.float32),
    )(parts, degp, x, W, b, res, W2, b2)


def kernel(t, y, edge_index, edge_type, W1, b1, rel1, W2, b2, rel2, res,
           Wmu, bmu):
    del t
    pad = E_PAD - N_EDGES
    src_p = jnp.concatenate(
        [edge_index[0], jnp.zeros((pad,), edge_index.dtype)]).astype(jnp.int32)
    et_p = jnp.concatenate(
        [edge_type, jnp.zeros((pad,), edge_type.dtype)]).astype(jnp.int32)
    dst_p = jnp.concatenate(
        [edge_index[1],
         N_NODES + (jnp.arange(pad, dtype=jnp.int32) % (NP - N_NODES))]
    ).astype(jnp.int32)

    idx = _edge_idx(src_p, et_p).reshape(NW, 2, HKCH, CHUNK)
    dst3 = dst_p.reshape(NW, 2, HKCH, CHUNK)

    zeros_big = jnp.zeros((NP, DIM), jnp.float32)

    ones_src = jnp.ones((CHUNK, DIM), jnp.float32)

    b1r = b1.reshape(1, DIM)
    b2r = b2.reshape(1, DIM)
    bmur = bmu.reshape(1, DIM)
    resr = res.reshape(1, 1)

    degp = _sc_deg(dst3, zeros_big, ones_src)
    xr1 = _build_xr(y, rel1)
    agg1 = _sc_edges(xr1, idx, dst3, zeros_big)
    emb1 = _post(False, agg1, degp, y, W1, b1r, resr, W1, b1r)

    xr2 = _build_xr(emb1, rel2)
    agg2 = _sc_edges(xr2, idx, dst3, zeros_big)
    out = _post(True, agg2, degp, emb1, W2, b2r, resr, Wmu, bmur)
    return out


# speedup vs baseline: 1.0568x; 1.0568x over previous
import functools

import jax
import jax.numpy as jnp
from jax import lax
from jax.experimental import pallas as pl
from jax.experimental.pallas import tpu as pltpu
from jax.experimental.pallas import tpu_sc as plsc

N_NODES = 10000
DIM = 128
N_EDGES = 320000
N_REL = 16

NC = 2
NS = 16
NW = NC * NS

CHUNK = 128
KCH = 80
HKCH = KCH // 2
PW = KCH * CHUNK
E_PAD = PW * NW
TCH = E_PAD // CHUNK
KA = 128
KB = 32
KMAXH = max(KA, KB) // 2
NP = 10112
STRIPE = NP // NS

_mesh = plsc.VectorSubcoreMesh(core_axis_name="c", subcore_axis_name="s",
                               num_cores=NC, num_subcores=NS)


def _sc_edge_body(xr_hbm, idx_hbm, dst_hbm, zeros_hbm, agg_out, agg_sh,
                  idx_v, dst_v, buf_a, buf_b, sem_a, sem_b):
    c = lax.axis_index("c")
    s = lax.axis_index("s")
    wid = c * NS + s

    row0 = s * STRIPE
    pltpu.sync_copy(zeros_hbm.at[pl.ds(row0, STRIPE)],
                    agg_sh.at[pl.ds(row0, STRIPE)])
    plsc.subcore_barrier()

    def gat(jv, buf, sem):
        pltpu.async_copy(xr_hbm.at[idx_v.at[jv]], buf, sem)

    def gwait(buf, sem):
        pltpu.make_async_copy(zeros_hbm.at[pl.ds(0, CHUNK)], buf, sem).wait()

    def sca(jv, buf):
        pltpu.sync_copy(buf, agg_sh.at[dst_v.at[jv]], add=True)

    def pipeline(k, start):
        kh = k // 2
        for h in range(2):
            base = start + h * kh
            pltpu.sync_copy(idx_hbm.at[pl.ds(base, kh)],
                            idx_v.at[pl.ds(0, kh)])
            pltpu.sync_copy(dst_hbm.at[pl.ds(base, kh)],
                            dst_v.at[pl.ds(0, kh)])
            gat(0, buf_a, sem_a)

            @pl.loop(0, kh // 2 - 1)
            def _(p):
                j = 2 * p
                gat(j + 1, buf_b, sem_b)
                gwait(buf_a, sem_a)
                sca(j, buf_a)
                gat(j + 2, buf_a, sem_a)
                gwait(buf_b, sem_b)
                sca(j + 1, buf_b)

            gat(kh - 1, buf_b, sem_b)
            gwait(buf_a, sem_a)
            sca(kh - 2, buf_a)
            gwait(buf_b, sem_b)
            sca(kh - 1, buf_b)

    @pl.when(c == 0)
    def _():
        pipeline(KA, s * KA)

    @pl.when(c == 1)
    def _():
        pipeline(KB, NS * KA + s * KB)

    plsc.subcore_barrier()
    pltpu.sync_copy(agg_sh.at[pl.ds(row0, STRIPE)],
                    agg_out.at[c, pl.ds(row0, STRIPE)])


_sc_edges = pl.kernel(
    _sc_edge_body,
    out_type=jax.ShapeDtypeStruct((NC, NP, DIM), jnp.float32),
    mesh=_mesh,
    scratch_types=(
        pltpu.VMEM_SHARED((NP, DIM), jnp.float32),
        pltpu.VMEM((KMAXH, CHUNK), jnp.int32),
        pltpu.VMEM((KMAXH, CHUNK), jnp.int32),
        pltpu.VMEM((CHUNK, DIM), jnp.float32),
        pltpu.VMEM((CHUNK, DIM), jnp.float32),
        pltpu.SemaphoreType.DMA,
        pltpu.SemaphoreType.DMA,
    ),
)


def _sc_deg_body(dst_hbm, zdeg_hbm, ones_hbm, deg_out, deg_sh, dst_v,
                 ones_v, sem):
    c = lax.axis_index("c")
    s = lax.axis_index("s")
    wid = c * NS + s

    pltpu.sync_copy(ones_hbm, ones_v)
    row0 = s * STRIPE
    pltpu.sync_copy(zdeg_hbm.at[pl.ds(row0, STRIPE)],
                    deg_sh.at[pl.ds(row0, STRIPE)])
    plsc.subcore_barrier()

    def swait():
        pltpu.make_async_copy(zdeg_hbm.at[pl.ds(0, CHUNK)], ones_v,
                              sem).wait()

    for h in range(2):
        pltpu.sync_copy(dst_hbm.at[pl.ds(wid * KCH + h * HKCH, HKCH)], dst_v)
        pltpu.async_copy(ones_v, deg_sh.at[dst_v.at[0]], sem, add=True)

        @pl.loop(1, HKCH)
        def _(j):
            pltpu.async_copy(ones_v, deg_sh.at[dst_v.at[j]], sem, add=True)
            swait()

        swait()

    plsc.subcore_barrier()
    pltpu.sync_copy(deg_sh.at[pl.ds(row0, STRIPE)],
                    deg_out.at[c, pl.ds(row0, STRIPE)])


_sc_deg = pl.kernel(
    _sc_deg_body,
    out_type=jax.ShapeDtypeStruct((NC, NP, DIM), jnp.float32),
    mesh=_mesh,
    scratch_types=(
        pltpu.VMEM_SHARED((NP, DIM), jnp.float32),
        pltpu.VMEM((HKCH, CHUNK), jnp.int32),
        pltpu.VMEM((CHUNK, DIM), jnp.float32),
        pltpu.SemaphoreType.DMA,
    ),
)


_EROWS = E_PAD // 128


def _eidx_body(src_ref, et_ref, out_ref):
    out_ref[...] = et_ref[...] * N_NODES + src_ref[...]


def _edge_idx(src_p, et_p):
    return pl.pallas_call(
        _eidx_body,
        out_shape=jax.ShapeDtypeStruct((_EROWS, 128), jnp.int32),
    )(src_p.reshape(_EROWS, 128), et_p.reshape(_EROWS, 128))


NB = 400


def _xr_body(x_ref, rel_ref, out_ref):
    out_ref[...] = rel_ref[...][:, None, :] * x_ref[...][None, :, :]


def _build_xr(x, rel):
    grid = (N_NODES // NB,)
    xr = pl.pallas_call(
        _xr_body,
        grid=grid,
        in_specs=[
            pl.BlockSpec((NB, DIM), lambda i: (i, 0)),
            pl.BlockSpec((N_REL, DIM), lambda i: (0, 0)),
        ],
        out_specs=pl.BlockSpec((N_REL, NB, DIM), lambda i: (0, i, 0)),
        out_shape=jax.ShapeDtypeStruct((N_REL, N_NODES, DIM), jnp.float32),
    )(x, rel)
    return xr.reshape(N_REL * N_NODES, DIM)


def _post_body(final, parts_ref, degp_ref, x_ref, W_ref, b_ref, res_ref,
               W2_ref, b2_ref, out_ref):
    deg = degp_ref[0, :, 0:1] + degp_ref[1, :, 0:1]
    deg = jnp.maximum(deg, 1.0)
    agg = (parts_ref[0] + parts_ref[1]) / deg
    h = jax.nn.relu(
        jnp.dot(agg, W_ref[...], preferred_element_type=jnp.float32)
        + b_ref[...])
    emb = x_ref[...] + res_ref[0, 0] * h
    if final:
        out_ref[...] = (
            jnp.dot(emb, W2_ref[...], preferred_element_type=jnp.float32)
            + b2_ref[...])
    else:
        out_ref[...] = emb


def _post(final, parts, degp, x, W, b, res, W2, b2):
    grid = (N_NODES // NB,)
    return pl.pallas_call(
        functools.partial(_post_body, final),
        grid=grid,
        in_specs=[
            pl.BlockSpec((NC, NB, DIM), lambda i: (0, i, 0)),
            pl.BlockSpec((NC, NB, DIM), lambda i: (0, i, 0)),
            pl.BlockSpec((NB, DIM), lambda i: (i, 0)),
            pl.BlockSpec((DIM, DIM), lambda i: (0, 0)),
            pl.BlockSpec((1, DIM), lambda i: (0, 0)),
            pl.BlockSpec((1, 1), lambda i: (0, 0)),
            pl.BlockSpec((DIM, DIM), lambda i: (0, 0)),
            pl.BlockSpec((1, DIM), lambda i: (0, 0)),
        ],
        out_specs=pl.BlockSpec((NB, DIM), lambda i: (i, 0)),
        out_shape=jax.ShapeDtypeStruct((N_NODES, DIM), jnp.float32),
    )(parts, degp, x, W, b, res, W2, b2)


def kernel(t, y, edge_index, edge_type, W1, b1, rel1, W2, b2, rel2, res,
           Wmu, bmu):
    del t
    pad = E_PAD - N_EDGES
    src_p = jnp.concatenate(
        [edge_index[0], jnp.zeros((pad,), edge_index.dtype)]).astype(jnp.int32)
    et_p = jnp.concatenate(
        [edge_type, jnp.zeros((pad,), edge_type.dtype)]).astype(jnp.int32)
    dst_p = jnp.concatenate(
        [edge_index[1],
         N_NODES + (jnp.arange(pad, dtype=jnp.int32) % (NP - N_NODES))]
    ).astype(jnp.int32)

    idx = _edge_idx(src_p, et_p)
    dst3 = dst_p.reshape(TCH, CHUNK)

    zeros_big = jnp.zeros((NP, DIM), jnp.float32)

    ones_src = jnp.ones((CHUNK, DIM), jnp.float32)

    b1r = b1.reshape(1, DIM)
    b2r = b2.reshape(1, DIM)
    bmur = bmu.reshape(1, DIM)
    resr = res.reshape(1, 1)

    degp = _sc_deg(dst3, zeros_big, ones_src)
    xr1 = _build_xr(y, rel1)
    agg1 = _sc_edges(xr1, idx, dst3, zeros_big)
    emb1 = _post(False, agg1, degp, y, W1, b1r, resr, W1, b1r)

    xr2 = _build_xr(emb1, rel2)
    agg2 = _sc_edges(xr2, idx, dst3, zeros_big)
    out = _post(True, agg2, degp, emb1, W2, b2r, resr, Wmu, bmur)
    return out


# speedup vs baseline: 2.7053x; 2.5600x over previous
import functools

import jax
import jax.numpy as jnp
from jax import lax
from jax.experimental import pallas as pl
from jax.experimental.pallas import tpu as pltpu
from jax.experimental.pallas import tpu_sc as plsc

N_NODES = 10000
DIM = 128
N_EDGES = 320000
N_REL = 16

NC = 2
NS = 16
NW = NC * NS

CHUNK = 128
KCH = 80
HKCH = KCH // 2
PW = KCH * CHUNK
E_PAD = PW * NW
TCH = E_PAD // CHUNK
KA = 80
KB = 80
KMAXH = max(KA, KB) // 2
NP = 10112
STRIPE = NP // NS

_mesh = plsc.VectorSubcoreMesh(core_axis_name="c", subcore_axis_name="s",
                               num_cores=NC, num_subcores=NS)


def _sc_edge_body(xr_hbm, idx_hbm, dst_hbm, zeros_hbm, agg_out, agg_sh,
                  idx_v, dst_v, buf_a, buf_b, sem_a, sem_b):
    c = lax.axis_index("c")
    s = lax.axis_index("s")
    wid = c * NS + s

    row0 = s * STRIPE
    pltpu.sync_copy(zeros_hbm.at[pl.ds(row0, STRIPE)],
                    agg_sh.at[pl.ds(row0, STRIPE)])
    plsc.subcore_barrier()

    def gat(jv, buf, sem):
        pltpu.async_copy(xr_hbm.at[idx_v.at[jv]], buf, sem)

    def gwait(buf, sem):
        pltpu.make_async_copy(zeros_hbm.at[pl.ds(0, CHUNK)], buf, sem).wait()

    def sca(jv, buf):
        pltpu.sync_copy(buf, agg_sh.at[dst_v.at[jv]], add=True)

    def pipeline(k, start):
        kh = k // 2
        for h in range(2):
            base = start + h * kh
            pltpu.sync_copy(idx_hbm.at[pl.ds(base, kh)],
                            idx_v.at[pl.ds(0, kh)])
            pltpu.sync_copy(dst_hbm.at[pl.ds(base, kh)],
                            dst_v.at[pl.ds(0, kh)])
            gat(0, buf_a, sem_a)

            @pl.loop(0, kh // 2 - 1)
            def _(p):
                j = 2 * p
                gat(j + 1, buf_b, sem_b)
                gwait(buf_a, sem_a)
                sca(j, buf_a)
                gat(j + 2, buf_a, sem_a)
                gwait(buf_b, sem_b)
                sca(j + 1, buf_b)

            gat(kh - 1, buf_b, sem_b)
            gwait(buf_a, sem_a)
            sca(kh - 2, buf_a)
            gwait(buf_b, sem_b)
            sca(kh - 1, buf_b)

    @pl.when(c == 0)
    def _():
        pipeline(KA, s * KA)

    @pl.when(c == 1)
    def _():
        pipeline(KB, NS * KA + s * KB)

    plsc.subcore_barrier()
    pltpu.sync_copy(agg_sh.at[pl.ds(row0, STRIPE)],
                    agg_out.at[c, pl.ds(row0, STRIPE)])


_sc_edges = pl.kernel(
    _sc_edge_body,
    out_type=jax.ShapeDtypeStruct((NC, NP, DIM), jnp.float32),
    mesh=_mesh,
    scratch_types=(
        pltpu.VMEM_SHARED((NP, DIM), jnp.float32),
        pltpu.VMEM((KMAXH, CHUNK), jnp.int32),
        pltpu.VMEM((KMAXH, CHUNK), jnp.int32),
        pltpu.VMEM((CHUNK, DIM), jnp.float32),
        pltpu.VMEM((CHUNK, DIM), jnp.float32),
        pltpu.SemaphoreType.DMA,
        pltpu.SemaphoreType.DMA,
    ),
)


def _sc_deg_body(dst_hbm, zdeg_hbm, ones_hbm, deg_out, deg_sh, dst_v,
                 ones_v, sem):
    c = lax.axis_index("c")
    s = lax.axis_index("s")
    wid = c * NS + s

    pltpu.sync_copy(ones_hbm, ones_v)
    row0 = s * STRIPE
    pltpu.sync_copy(zdeg_hbm.at[pl.ds(row0, STRIPE)],
                    deg_sh.at[pl.ds(row0, STRIPE)])
    plsc.subcore_barrier()

    def swait():
        pltpu.make_async_copy(zdeg_hbm.at[pl.ds(0, CHUNK)], ones_v,
                              sem).wait()

    for h in range(2):
        pltpu.sync_copy(dst_hbm.at[pl.ds(wid * KCH + h * HKCH, HKCH)], dst_v)
        pltpu.async_copy(ones_v, deg_sh.at[dst_v.at[0]], sem, add=True)

        @pl.loop(1, HKCH)
        def _(j):
            pltpu.async_copy(ones_v, deg_sh.at[dst_v.at[j]], sem, add=True)
            swait()

        swait()

    plsc.subcore_barrier()
    pltpu.sync_copy(deg_sh.at[pl.ds(row0, STRIPE)],
                    deg_out.at[c, pl.ds(row0, STRIPE)])


_sc_deg = pl.kernel(
    _sc_deg_body,
    out_type=jax.ShapeDtypeStruct((NC, NP, DIM), jnp.float32),
    mesh=_mesh,
    scratch_types=(
        pltpu.VMEM_SHARED((NP, DIM), jnp.float32),
        pltpu.VMEM((HKCH, CHUNK), jnp.int32),
        pltpu.VMEM((CHUNK, DIM), jnp.float32),
        pltpu.SemaphoreType.DMA,
    ),
)


_EROWS = E_PAD // 128


def _eidx_body(src_ref, et_ref, out_ref):
    out_ref[...] = et_ref[...] * N_NODES + src_ref[...]


def _edge_idx(src_p, et_p):
    return pl.pallas_call(
        _eidx_body,
        out_shape=jax.ShapeDtypeStruct((_EROWS, 128), jnp.int32),
    )(src_p.reshape(_EROWS, 128), et_p.reshape(_EROWS, 128))


NB = 400


def _xr_body(x_ref, rel_ref, out_ref):
    out_ref[...] = rel_ref[...][:, None, :] * x_ref[...][None, :, :]


def _build_xr(x, rel):
    grid = (N_NODES // NB,)
    xr = pl.pallas_call(
        _xr_body,
        grid=grid,
        in_specs=[
            pl.BlockSpec((NB, DIM), lambda i: (i, 0)),
            pl.BlockSpec((N_REL, DIM), lambda i: (0, 0)),
        ],
        out_specs=pl.BlockSpec((N_REL, NB, DIM), lambda i: (0, i, 0)),
        out_shape=jax.ShapeDtypeStruct((N_REL, N_NODES, DIM), jnp.float32),
    )(x, rel)
    return xr.reshape(N_REL * N_NODES, DIM)


def _post_body(final, parts_ref, degp_ref, x_ref, W_ref, b_ref, res_ref,
               W2_ref, b2_ref, out_ref):
    deg = degp_ref[0, :, 0:1] + degp_ref[1, :, 0:1]
    deg = jnp.maximum(deg, 1.0)
    agg = (parts_ref[0] + parts_ref[1]) / deg
    h = jax.nn.relu(
        jnp.dot(agg, W_ref[...], preferred_element_type=jnp.float32)
        + b_ref[...])
    emb = x_ref[...] + res_ref[0, 0] * h
    if final:
        out_ref[...] = (
            jnp.dot(emb, W2_ref[...], preferred_element_type=jnp.float32)
            + b2_ref[...])
    else:
        out_ref[...] = emb


def _post(final, parts, degp, x, W, b, res, W2, b2):
    grid = (N_NODES // NB,)
    return pl.pallas_call(
        functools.partial(_post_body, final),
        grid=grid,
        in_specs=[
            pl.BlockSpec((NC, NB, DIM), lambda i: (0, i, 0)),
            pl.BlockSpec((NC, NB, DIM), lambda i: (0, i, 0)),
            pl.BlockSpec((NB, DIM), lambda i: (i, 0)),
            pl.BlockSpec((DIM, DIM), lambda i: (0, 0)),
            pl.BlockSpec((1, DIM), lambda i: (0, 0)),
            pl.BlockSpec((1, 1), lambda i: (0, 0)),
            pl.BlockSpec((DIM, DIM), lambda i: (0, 0)),
            pl.BlockSpec((1, DIM), lambda i: (0, 0)),
        ],
        out_specs=pl.BlockSpec((NB, DIM), lambda i: (i, 0)),
        out_shape=jax.ShapeDtypeStruct((N_NODES, DIM), jnp.float32),
    )(parts, degp, x, W, b, res, W2, b2)


def kernel(t, y, edge_index, edge_type, W1, b1, rel1, W2, b2, rel2, res,
           Wmu, bmu):
    del t
    pad = E_PAD - N_EDGES
    src_p = jnp.concatenate(
        [edge_index[0],
         jnp.arange(pad, dtype=jnp.int32) % N_NODES]).astype(jnp.int32)
    et_p = jnp.concatenate(
        [edge_type, jnp.zeros((pad,), edge_type.dtype)]).astype(jnp.int32)
    dst_p = jnp.concatenate(
        [edge_index[1],
         N_NODES + (jnp.arange(pad, dtype=jnp.int32) % (NP - N_NODES))]
    ).astype(jnp.int32)

    idx = _edge_idx(src_p, et_p)
    dst3 = dst_p.reshape(TCH, CHUNK)

    zeros_big = jnp.zeros((NP, DIM), jnp.float32)

    ones_src = jnp.ones((CHUNK, DIM), jnp.float32)

    b1r = b1.reshape(1, DIM)
    b2r = b2.reshape(1, DIM)
    bmur = bmu.reshape(1, DIM)
    resr = res.reshape(1, 1)

    degp = _sc_deg(dst3, zeros_big, ones_src)
    xr1 = _build_xr(y, rel1)
    agg1 = _sc_edges(xr1, idx, dst3, zeros_big)
    emb1 = _post(False, agg1, degp, y, W1, b1r, resr, W1, b1r)

    xr2 = _build_xr(emb1, rel2)
    agg2 = _sc_edges(xr2, idx, dst3, zeros_big)
    out = _post(True, agg2, degp, emb1, W2, b2r, resr, Wmu, bmur)
    return out


# speedup vs baseline: 2.8009x; 1.0353x over previous
import functools

import jax
import jax.numpy as jnp
from jax import lax
from jax.experimental import pallas as pl
from jax.experimental.pallas import tpu as pltpu
from jax.experimental.pallas import tpu_sc as plsc

N_NODES = 10000
DIM = 128
N_EDGES = 320000
N_REL = 16

NC = 2
NS = 16
NW = NC * NS

CHUNK = 128
KCH = 80
HKCH = KCH // 2
PW = KCH * CHUNK
E_PAD = PW * NW
TCH = E_PAD // CHUNK
KA = 80
KB = 80
KMAXH = max(KA, KB) // 2
NP = 10112
STRIPE = NP // NS

_mesh = plsc.VectorSubcoreMesh(core_axis_name="c", subcore_axis_name="s",
                               num_cores=NC, num_subcores=NS)


def _sc_edge_body(xr_hbm, idx_hbm, dst_hbm, zeros_hbm, agg_out, agg_sh,
                  idx_v, dst_v, buf_a, buf_b, sem_a, sem_b):
    c = lax.axis_index("c")
    s = lax.axis_index("s")
    wid = c * NS + s

    row0 = s * STRIPE
    pltpu.sync_copy(zeros_hbm.at[pl.ds(row0, STRIPE)],
                    agg_sh.at[pl.ds(row0, STRIPE)])
    plsc.subcore_barrier()

    def gat(jv, buf, sem):
        pltpu.async_copy(xr_hbm.at[idx_v.at[jv]], buf, sem)

    def gwait(buf, sem):
        pltpu.make_async_copy(zeros_hbm.at[pl.ds(0, CHUNK)], buf, sem).wait()

    def sca(jv, buf):
        pltpu.sync_copy(buf, agg_sh.at[dst_v.at[jv]], add=True)

    def pipeline(k, start):
        kh = k // 2
        for h in range(2):
            base = start + h * kh
            pltpu.sync_copy(idx_hbm.at[pl.ds(base, kh)],
                            idx_v.at[pl.ds(0, kh)])
            pltpu.sync_copy(dst_hbm.at[pl.ds(base, kh)],
                            dst_v.at[pl.ds(0, kh)])
            gat(0, buf_a, sem_a)

            @pl.loop(0, kh // 2 - 1)
            def _(p):
                j = 2 * p
                gat(j + 1, buf_b, sem_b)
                gwait(buf_a, sem_a)
                sca(j, buf_a)
                gat(j + 2, buf_a, sem_a)
                gwait(buf_b, sem_b)
                sca(j + 1, buf_b)

            gat(kh - 1, buf_b, sem_b)
            gwait(buf_a, sem_a)
            sca(kh - 2, buf_a)
            gwait(buf_b, sem_b)
            sca(kh - 1, buf_b)

    @pl.when(c == 0)
    def _():
        pipeline(KA, s * KA)

    @pl.when(c == 1)
    def _():
        pipeline(KB, NS * KA + s * KB)

    plsc.subcore_barrier()
    pltpu.sync_copy(agg_sh.at[pl.ds(row0, STRIPE)],
                    agg_out.at[c, pl.ds(row0, STRIPE)])


_sc_edges = pl.kernel(
    _sc_edge_body,
    out_type=jax.ShapeDtypeStruct((NC, NP, DIM), jnp.float32),
    mesh=_mesh,
    scratch_types=(
        pltpu.VMEM_SHARED((NP, DIM), jnp.float32),
        pltpu.VMEM((KMAXH, CHUNK), jnp.int32),
        pltpu.VMEM((KMAXH, CHUNK), jnp.int32),
        pltpu.VMEM((CHUNK, DIM), jnp.float32),
        pltpu.VMEM((CHUNK, DIM), jnp.float32),
        pltpu.SemaphoreType.DMA,
        pltpu.SemaphoreType.DMA,
    ),
)


def _sc_deg_body(dst_hbm, zdeg_hbm, ones_hbm, deg_out, deg_sh, dst_v,
                 ones_v, sem):
    c = lax.axis_index("c")
    s = lax.axis_index("s")
    wid = c * NS + s

    pltpu.sync_copy(ones_hbm, ones_v)
    row0 = s * STRIPE
    pltpu.sync_copy(zdeg_hbm.at[pl.ds(row0, STRIPE)],
                    deg_sh.at[pl.ds(row0, STRIPE)])
    plsc.subcore_barrier()

    def swait():
        pltpu.make_async_copy(zdeg_hbm.at[pl.ds(0, CHUNK)], ones_v,
                              sem).wait()

    for h in range(2):
        pltpu.sync_copy(dst_hbm.at[pl.ds(wid * KCH + h * HKCH, HKCH)], dst_v)
        pltpu.async_copy(ones_v, deg_sh.at[dst_v.at[0]], sem, add=True)

        @pl.loop(1, HKCH)
        def _(j):
            pltpu.async_copy(ones_v, deg_sh.at[dst_v.at[j]], sem, add=True)
            swait()

        swait()

    plsc.subcore_barrier()
    pltpu.sync_copy(deg_sh.at[pl.ds(row0, STRIPE)],
                    deg_out.at[c, pl.ds(row0, STRIPE)])


_sc_deg = pl.kernel(
    _sc_deg_body,
    out_type=jax.ShapeDtypeStruct((NC, NP, DIM), jnp.float32),
    mesh=_mesh,
    scratch_types=(
        pltpu.VMEM_SHARED((NP, DIM), jnp.float32),
        pltpu.VMEM((HKCH, CHUNK), jnp.int32),
        pltpu.VMEM((CHUNK, DIM), jnp.float32),
        pltpu.SemaphoreType.DMA,
    ),
)


_EROWS = E_PAD // 128


def _eidx_body(src_ref, et_ref, out_ref):
    out_ref[...] = et_ref[...] * N_NODES + src_ref[...]


def _edge_idx(src_p, et_p):
    return pl.pallas_call(
        _eidx_body,
        out_shape=jax.ShapeDtypeStruct((_EROWS, 128), jnp.int32),
    )(src_p.reshape(_EROWS, 128), et_p.reshape(_EROWS, 128))


NB = 400


def _xr_body(x_ref, rel_ref, out_ref):
    out_ref[...] = rel_ref[...][:, None, :] * x_ref[...][None, :, :]


def _build_xr(x, rel):
    grid = (N_NODES // NB,)
    xr = pl.pallas_call(
        _xr_body,
        grid=grid,
        in_specs=[
            pl.BlockSpec((NB, DIM), lambda i: (i, 0)),
            pl.BlockSpec((N_REL, DIM), lambda i: (0, 0)),
        ],
        out_specs=pl.BlockSpec((N_REL, NB, DIM), lambda i: (0, i, 0)),
        out_shape=jax.ShapeDtypeStruct((N_REL, N_NODES, DIM), jnp.float32),
    )(x, rel)
    return xr.reshape(N_REL * N_NODES, DIM)


def _post_body(final, parts_ref, degp_ref, x_ref, W_ref, b_ref, res_ref,
               W2_ref, b2_ref, out_ref, xr_ref):
    deg = degp_ref[0, :, 0:1] + degp_ref[1, :, 0:1]
    deg = jnp.maximum(deg, 1.0)
    agg = (parts_ref[0] + parts_ref[1]) / deg
    h = jax.nn.relu(
        jnp.dot(agg, W_ref[...], preferred_element_type=jnp.float32)
        + b_ref[...])
    emb = x_ref[...] + res_ref[0, 0] * h
    if final:
        out_ref[...] = (
            jnp.dot(emb, W2_ref[...], preferred_element_type=jnp.float32)
            + b2_ref[...])
    else:
        out_ref[...] = emb
        xr_ref[...] = W2_ref[...][:N_REL][:, None, :] * emb[None, :, :]


def _post(final, parts, degp, x, W, b, res, W2, b2):
    grid = (N_NODES // NB,)
    out_specs = [pl.BlockSpec((NB, DIM), lambda i: (i, 0))]
    out_shape = [jax.ShapeDtypeStruct((N_NODES, DIM), jnp.float32)]
    if not final:
        out_specs.append(pl.BlockSpec((N_REL, NB, DIM), lambda i: (0, i, 0)))
        out_shape.append(
            jax.ShapeDtypeStruct((N_REL, N_NODES, DIM), jnp.float32))

    def body(*refs):
        if final:
            _post_body(final, *refs, None)
        else:
            _post_body(final, *refs)

    return pl.pallas_call(
        body,
        grid=grid,
        in_specs=[
            pl.BlockSpec((NC, NB, DIM), lambda i: (0, i, 0)),
            pl.BlockSpec((NC, NB, DIM), lambda i: (0, i, 0)),
            pl.BlockSpec((NB, DIM), lambda i: (i, 0)),
            pl.BlockSpec((DIM, DIM), lambda i: (0, 0)),
            pl.BlockSpec((1, DIM), lambda i: (0, 0)),
            pl.BlockSpec((1, 1), lambda i: (0, 0)),
            pl.BlockSpec((DIM, DIM), lambda i: (0, 0)),
            pl.BlockSpec((1, DIM), lambda i: (0, 0)),
        ],
        out_specs=out_specs,
        out_shape=out_shape,
    )(parts, degp, x, W, b, res, W2, b2)


def kernel(t, y, edge_index, edge_type, W1, b1, rel1, W2, b2, rel2, res,
           Wmu, bmu):
    del t
    pad = E_PAD - N_EDGES
    src_p = jnp.concatenate(
        [edge_index[0],
         jnp.arange(pad, dtype=jnp.int32) % N_NODES]).astype(jnp.int32)
    et_p = jnp.concatenate(
        [edge_type, jnp.zeros((pad,), edge_type.dtype)]).astype(jnp.int32)
    dst_p = jnp.concatenate(
        [edge_index[1],
         N_NODES + (jnp.arange(pad, dtype=jnp.int32) % (NP - N_NODES))]
    ).astype(jnp.int32)

    idx = _edge_idx(src_p, et_p)
    dst3 = dst_p.reshape(TCH, CHUNK)

    zeros_big = jnp.zeros((NP, DIM), jnp.float32)

    ones_src = jnp.ones((CHUNK, DIM), jnp.float32)

    b1r = b1.reshape(1, DIM)
    b2r = b2.reshape(1, DIM)
    bmur = bmu.reshape(1, DIM)
    resr = res.reshape(1, 1)

    degp = _sc_deg(dst3, zeros_big, ones_src)
    xr1 = _build_xr(y, rel1)
    agg1 = _sc_edges(xr1, idx, dst3, zeros_big)
    rel2_pad = jnp.zeros((DIM, DIM), jnp.float32).at[:N_REL].set(rel2)
    emb1, xr2 = _post(False, agg1, degp, y, W1, b1r, resr, rel2_pad, b1r)

    agg2 = _sc_edges(xr2.reshape(N_REL * N_NODES, DIM), idx, dst3, zeros_big)
    (out,) = _post(True, agg2, degp, emb1, W2, b2r, resr, Wmu, bmur)
    return out
